# Initial kernel scaffold; baseline (speedup 1.0000x reference)
import jax
import jax.numpy as jnp
from jax.experimental import pallas as pl


def kernel(x, edge_index, edge_attr, We, be, W1, We1, a_s1, a_d1, a_e1, b1, W2, We2, a_s2, a_d2, a_e2, b2, Ws1, bs1, Ws2, bs2):
    raise NotImplementedError("write your pallas kernel here")


# speedup vs baseline: 6.4010x; 6.4010x over previous
import functools

import jax
import jax.numpy as jnp
from jax import lax
from jax.experimental import pallas as pl
from jax.experimental.pallas import tpu as pltpu
from jax.experimental.pallas import tpu_sc as plsc

N = 10000
E = 320000
D = 128
DE = 16
H = 64

NP = 10240
C = 512
NCH = E // C
NWK = 32
MAXCH = -(-NCH // NWK)
RPT = NP // 16

f32 = jnp.float32
i32 = jnp.int32

_HIGH = lax.Precision.HIGHEST

_mesh = plsc.VectorSubcoreMesh(core_axis_name="c", subcore_axis_name="s")
_sc_params = pltpu.CompilerParams(needs_layout_passes=False,
                                  use_tc_tiling_on_sc=False)


def _tc0_body(x_ref, w1_ref, a1_ref, we_ref, be_ref, we1_ref, ae1_ref,
              we2_ref, ae2_ref, h_ref, sd_ref, ubc1_ref, ubc2_ref, cbc_ref):
    h = jnp.dot(x_ref[...], w1_ref[...], precision=_HIGH)
    h_ref[...] = h
    sd_ref[...] = jnp.dot(h, a1_ref[...], precision=_HIGH)
    v1 = jnp.dot(we1_ref[...], ae1_ref[...], precision=_HIGH)
    v2 = jnp.dot(we2_ref[...], ae2_ref[...], precision=_HIGH)
    u1 = jnp.dot(we_ref[...], v1, precision=_HIGH)
    u2 = jnp.dot(we_ref[...], v2, precision=_HIGH)
    ubc1_ref[...] = jnp.broadcast_to(u1, (DE, 16))
    ubc2_ref[...] = jnp.broadcast_to(u2, (DE, 16))
    c1 = jnp.dot(be_ref[...], v1, precision=_HIGH)
    c2 = jnp.dot(be_ref[...], v2, precision=_HIGH)
    cbc_ref[...] = jnp.concatenate(
        [jnp.broadcast_to(c1, (1, 16)), jnp.broadcast_to(c2, (1, 16))], axis=0)


_tc0 = pl.pallas_call(
    _tc0_body,
    out_shape=(
        jax.ShapeDtypeStruct((NP, H), f32),
        jax.ShapeDtypeStruct((NP, 2), f32),
        jax.ShapeDtypeStruct((DE, 16), f32),
        jax.ShapeDtypeStruct((DE, 16), f32),
        jax.ShapeDtypeStruct((2, 16), f32),
    ),
)


def _tc1_body(o_ref, b_ref, w2_ref, a2_ref, h2_ref, sd2_ref):
    x1 = jnp.maximum(o_ref[0] + o_ref[1] + b_ref[...], 0.0)
    h2 = jnp.dot(x1, w2_ref[...], precision=_HIGH)
    h2_ref[...] = h2
    sd2_ref[...] = jnp.dot(h2, a2_ref[...], precision=_HIGH)


_tc1 = pl.pallas_call(
    _tc1_body,
    out_shape=(
        jax.ShapeDtypeStruct((NP, H), f32),
        jax.ShapeDtypeStruct((NP, 2), f32),
    ),
)


def _tc2_body(o_ref, b_ref, ws1_ref, bs1_ref, ws2_ref, bs2_ref,
              gs_ref, gd_ref, w2bc_ref, b2bc_ref):
    x2 = o_ref[0] + o_ref[1] + b_ref[...]
    gs_ref[...] = jnp.dot(x2, ws1_ref[0:H, :], precision=_HIGH)
    gd_ref[...] = jnp.dot(x2, ws1_ref[H:2 * H, :], precision=_HIGH) + bs1_ref[...]
    w2bc_ref[...] = jnp.broadcast_to(ws2_ref[...], (H, 16))
    b2bc_ref[...] = jnp.broadcast_to(bs2_ref[...], (1, 16))


_tc2 = pl.pallas_call(
    _tc2_body,
    out_shape=(
        jax.ShapeDtypeStruct((NP, H), f32),
        jax.ShapeDtypeStruct((NP, H), f32),
        jax.ShapeDtypeStruct((H, 16), f32),
        jax.ShapeDtypeStruct((1, 16), f32),
    ),
)


@functools.partial(
    pl.kernel,
    out_type=(
        jax.ShapeDtypeStruct((E,), f32),
        jax.ShapeDtypeStruct((2, NP), f32),
    ),
    mesh=_mesh,
    compiler_params=_sc_params,
    scratch_types=[
        pltpu.VMEM((NP, 2), f32),
        pltpu.VMEM((DE, 16), f32),
        pltpu.VMEM((1, 16), f32),
        pltpu.VMEM((C,), i32),
        pltpu.VMEM((C,), i32),
        pltpu.VMEM((C, DE), f32),
        pltpu.VMEM((C,), f32),
        pltpu.VMEM((NP,), f32),
        pltpu.VMEM_SHARED((16, NP), f32),
        pltpu.VMEM((16, RPT), f32),
        pltpu.VMEM((RPT,), f32),
    ],
)
def _pass_a(ei_hbm, ea_hbm, sd_hbm, ubc_hbm, cb_hbm, p_hbm, dn_hbm,
            sd_v, ubc_v, cb_v, src_v, dst_v, ea_v, p_v, dloc, dsh, red, outb):
    c = lax.axis_index("c")
    s = lax.axis_index("s")
    w = s * 2 + c
    pltpu.sync_copy(sd_hbm, sd_v)
    pltpu.sync_copy(ubc_hbm, ubc_v)
    pltpu.sync_copy(cb_hbm, cb_v)
    cbv = cb_v[0]
    uvs = [ubc_v[k] for k in range(DE)]
    iota = lax.iota(i32, 16)
    zeros16 = jnp.zeros((16,), f32)
    col0 = jnp.zeros((16,), i32)
    col1 = jnp.full((16,), 1, i32)

    @pl.loop(0, NP, step=16)
    def _(i):
        dloc[pl.ds(i, 16)] = zeros16

    @pl.loop(0, MAXCH)
    def _(it):
        ci = w + it * NWK

        @pl.when(ci < NCH)
        def _():
            off = ci * C
            pltpu.sync_copy(ei_hbm.at[0, pl.ds(off, C)], src_v)
            pltpu.sync_copy(ei_hbm.at[1, pl.ds(off, C)], dst_v)
            pltpu.sync_copy(ea_hbm.at[pl.ds(off, C), :], ea_v)

            @pl.loop(0, C, step=16)
            def _(g):
                sv = src_v[pl.ds(g, 16)]
                dv = dst_v[pl.ds(g, 16)]
                ev = iota + g
                acc = cbv
                for k in range(DE):
                    kc = jnp.full((16,), k, i32)
                    acc = acc + plsc.load_gather(ea_v, [ev, kc]) * uvs[k]
                a_s = plsc.load_gather(sd_v, [sv, col0])
                a_d = plsc.load_gather(sd_v, [dv, col1])
                lg = a_s + a_d + acc
                lg = jnp.maximum(lg, lg * 0.2)
                pe = jnp.exp(lg)
                p_v[pl.ds(g, 16)] = pe
                plsc.addupdate_scatter(dloc, [dv], pe)

            pltpu.sync_copy(p_v, p_hbm.at[pl.ds(off, C)])

    pltpu.sync_copy(dloc, dsh.at[s])
    plsc.subcore_barrier()
    pltpu.sync_copy(dsh.at[:, pl.ds(s * RPT, RPT)], red)
    for j in range(0, RPT, 16):
        acc = red[0, pl.ds(j, 16)]
        for t in range(1, 16):
            acc = acc + red[t, pl.ds(j, 16)]
        outb[pl.ds(j, 16)] = acc
    pltpu.sync_copy(outb, dn_hbm.at[c, pl.ds(s * RPT, RPT)])


@functools.partial(
    pl.kernel,
    out_type=jax.ShapeDtypeStruct((2, NP, H), f32),
    mesh=_mesh,
    compiler_params=_sc_params,
    scratch_types=[
        pltpu.VMEM((NP,), f32),
        pltpu.VMEM((NP,), f32),
        pltpu.VMEM((4, 128), i32),
        pltpu.VMEM((4, 128), i32),
        pltpu.VMEM((C,), i32),
        pltpu.VMEM((C,), f32),
        pltpu.VMEM((C, H), f32),
        pltpu.VMEM_SHARED((NP, H), f32),
    ],
)
def _pass_b(ei_hbm, p_hbm, dn_hbm, h_hbm, out_hbm,
            dtab, d2, src2, dst2, dstf, p_v, rowbuf, osh):
    c = lax.axis_index("c")
    s = lax.axis_index("s")
    w = s * 2 + c
    iota = lax.iota(i32, 16)
    zeros16 = jnp.zeros((16,), f32)

    pltpu.sync_copy(dn_hbm.at[0], dtab)
    pltpu.sync_copy(dn_hbm.at[1], d2)

    @pl.loop(0, NP, step=16)
    def _(i):
        dtab[pl.ds(i, 16)] = dtab[pl.ds(i, 16)] + d2[pl.ds(i, 16)]

    @pl.loop(0, 320)
    def _(r):
        for kk in range(H // 16):
            rowbuf[r, pl.ds(kk * 16, 16)] = zeros16

    pltpu.sync_copy(rowbuf.at[pl.ds(0, 320), :], osh.at[pl.ds(s * RPT, 320), :])
    pltpu.sync_copy(rowbuf.at[pl.ds(0, 320), :],
                    osh.at[pl.ds(s * RPT + 320, 320), :])
    plsc.subcore_barrier()

    @pl.loop(0, MAXCH)
    def _(it):
        ci = w + it * NWK

        @pl.when(ci < NCH)
        def _():
            off = ci * C
            for j in range(4):
                pltpu.sync_copy(ei_hbm.at[0, pl.ds(off + j * 128, 128)],
                                src2.at[j])
                pltpu.sync_copy(ei_hbm.at[1, pl.ds(off + j * 128, 128)],
                                dst2.at[j])
            pltpu.sync_copy(ei_hbm.at[1, pl.ds(off, C)], dstf)
            pltpu.sync_copy(p_hbm.at[pl.ds(off, C)], p_v)
            for j in range(4):
                pltpu.sync_copy(h_hbm.at[src2.at[j]],
                                rowbuf.at[pl.ds(j * 128, 128), :])

            @pl.loop(0, C, step=16)
            def _(g):
                dv = dstf[pl.ds(g, 16)]
                pe = p_v[pl.ds(g, 16)]
                dn = plsc.load_gather(dtab, [dv])
                cf = pe / (dn + 1e-16)
                ev = iota + g
                for k in range(H):
                    kc = jnp.full((16,), k, i32)
                    v = plsc.load_gather(rowbuf, [ev, kc])
                    plsc.store_scatter(rowbuf, [ev, kc], v * cf)

            for j in range(4):
                pltpu.sync_copy(rowbuf.at[pl.ds(j * 128, 128), :],
                                osh.at[dst2.at[j]], add=True)

    plsc.subcore_barrier()
    pltpu.sync_copy(osh.at[pl.ds(s * RPT, 320), :], rowbuf.at[pl.ds(0, 320), :])
    pltpu.sync_copy(rowbuf.at[pl.ds(0, 320), :],
                    out_hbm.at[c, pl.ds(s * RPT, 320), :])
    pltpu.sync_copy(osh.at[pl.ds(s * RPT + 320, 320), :],
                    rowbuf.at[pl.ds(0, 320), :])
    pltpu.sync_copy(rowbuf.at[pl.ds(0, 320), :],
                    out_hbm.at[c, pl.ds(s * RPT + 320, 320), :])


@functools.partial(
    pl.kernel,
    out_type=jax.ShapeDtypeStruct((E,), f32),
    mesh=_mesh,
    compiler_params=_sc_params,
    scratch_types=[
        pltpu.VMEM((4, 128), i32),
        pltpu.VMEM((4, 128), i32),
        pltpu.VMEM((C, H), f32),
        pltpu.VMEM((C, H), f32),
        pltpu.VMEM((H, 16), f32),
        pltpu.VMEM((1, 16), f32),
        pltpu.VMEM((C,), f32),
    ],
)
def _pass_c(ei_hbm, gs_hbm, gd_hbm, w2bc_hbm, b2bc_hbm, sc_hbm,
            src2, dst2, sbuf, dbuf, w2_v, b2_v, sc_v):
    c = lax.axis_index("c")
    s = lax.axis_index("s")
    w = s * 2 + c
    iota = lax.iota(i32, 16)
    pltpu.sync_copy(w2bc_hbm, w2_v)
    pltpu.sync_copy(b2bc_hbm, b2_v)
    bias = b2_v[0]

    @pl.loop(0, MAXCH)
    def _(it):
        ci = w + it * NWK

        @pl.when(ci < NCH)
        def _():
            off = ci * C
            for j in range(4):
                pltpu.sync_copy(ei_hbm.at[0, pl.ds(off + j * 128, 128)],
                                src2.at[j])
                pltpu.sync_copy(ei_hbm.at[1, pl.ds(off + j * 128, 128)],
                                dst2.at[j])
            for j in range(4):
                pltpu.sync_copy(gs_hbm.at[src2.at[j]],
                                sbuf.at[pl.ds(j * 128, 128), :])
                pltpu.sync_copy(gd_hbm.at[dst2.at[j]],
                                dbuf.at[pl.ds(j * 128, 128), :])

            @pl.loop(0, C, step=16)
            def _(g):
                ev = iota + g
                acc = bias
                for k in range(H):
                    kc = jnp.full((16,), k, i32)
                    t = (plsc.load_gather(sbuf, [ev, kc])
                         + plsc.load_gather(dbuf, [ev, kc]))
                    acc = acc + jnp.maximum(t, 0.0) * w2_v[k]
                sc_v[pl.ds(g, 16)] = 1.0 / (1.0 + jnp.exp(-acc))

            pltpu.sync_copy(sc_v, sc_hbm.at[pl.ds(off, C)])


def kernel(x, edge_index, edge_attr, We, be, W1, We1, a_s1, a_d1, a_e1, b1,
           W2, We2, a_s2, a_d2, a_e2, b2, Ws1, bs1, Ws2, bs2):
    x_p = jnp.pad(x, ((0, NP - N), (0, 0)))
    A1 = jnp.stack([a_s1, a_d1], axis=1)
    A2 = jnp.stack([a_s2, a_d2], axis=1)

    h1, sd1, ubc1, ubc2, cbc = _tc0(
        x_p, W1, A1, We, be[None, :], We1, a_e1[:, None], We2, a_e2[:, None])

    p1, dn1 = _pass_a(edge_index, edge_attr, sd1, ubc1, cbc[0:1])
    o1 = _pass_b(edge_index, p1, dn1, h1)

    h2, sd2 = _tc1(o1, b1[None, :], W2, A2)

    p2, dn2 = _pass_a(edge_index, edge_attr, sd2, ubc2, cbc[1:2])
    o2 = _pass_b(edge_index, p2, dn2, h2)

    gs, gd, w2bc, b2bc = _tc2(
        o2, b2[None, :], Ws1, bs1[None, :], Ws2, bs2[:, None])

    return _pass_c(edge_index, gs, gd, w2bc, b2bc)


# speedup vs baseline: 8.0656x; 1.2601x over previous
import functools

import jax
import jax.numpy as jnp
from jax import lax
from jax.experimental import pallas as pl
from jax.experimental.pallas import tpu as pltpu
from jax.experimental.pallas import tpu_sc as plsc

N = 10000
E = 320000
D = 128
DE = 16
H = 64

NP = 10240
NWK = 32
RPT = NP // 16

C = 512
NCH = E // C
SAB = NCH // NWK
XAB = NCH - SAB * NWK

CC = 256
NCHC = E // CC
SC_ = NCHC // NWK
XC = NCHC - SC_ * NWK

ER = E // 128

f32 = jnp.float32
i32 = jnp.int32

_HIGH = lax.Precision.HIGHEST

_mesh = plsc.VectorSubcoreMesh(core_axis_name="c", subcore_axis_name="s")
_sc_params = pltpu.CompilerParams(needs_layout_passes=False,
                                  use_tc_tiling_on_sc=False)


def _tc0_body(x_ref, w1_ref, a1_ref, we_ref, be_ref, we1_ref, ae1_ref,
              we2_ref, ae2_ref, h_ref, sd_ref, ubc1_ref, ubc2_ref, cbc_ref):
    h = jnp.dot(x_ref[...], w1_ref[...], precision=_HIGH)
    h_ref[...] = h
    sd_ref[...] = jnp.dot(h, a1_ref[...], precision=_HIGH)
    v1 = jnp.dot(we1_ref[...], ae1_ref[...], precision=_HIGH)
    v2 = jnp.dot(we2_ref[...], ae2_ref[...], precision=_HIGH)
    u1 = jnp.dot(we_ref[...], v1, precision=_HIGH)
    u2 = jnp.dot(we_ref[...], v2, precision=_HIGH)
    ubc1_ref[...] = jnp.broadcast_to(u1, (DE, 16))
    ubc2_ref[...] = jnp.broadcast_to(u2, (DE, 16))
    c1 = jnp.dot(be_ref[...], v1, precision=_HIGH)
    c2 = jnp.dot(be_ref[...], v2, precision=_HIGH)
    cbc_ref[...] = jnp.concatenate(
        [jnp.broadcast_to(c1, (1, 16)), jnp.broadcast_to(c2, (1, 16))], axis=0)


_tc0 = pl.pallas_call(
    _tc0_body,
    out_shape=(
        jax.ShapeDtypeStruct((NP, H), f32),
        jax.ShapeDtypeStruct((NP, 2), f32),
        jax.ShapeDtypeStruct((DE, 16), f32),
        jax.ShapeDtypeStruct((DE, 16), f32),
        jax.ShapeDtypeStruct((2, 16), f32),
    ),
)


def _tc1_body(o_ref, b_ref, w2_ref, a2_ref, h2_ref, sd2_ref):
    x1 = jnp.maximum(o_ref[0] + o_ref[1] + b_ref[...], 0.0)
    h2 = jnp.dot(x1, w2_ref[...], precision=_HIGH)
    h2_ref[...] = h2
    sd2_ref[...] = jnp.dot(h2, a2_ref[...], precision=_HIGH)


_tc1 = pl.pallas_call(
    _tc1_body,
    out_shape=(
        jax.ShapeDtypeStruct((NP, H), f32),
        jax.ShapeDtypeStruct((NP, 2), f32),
    ),
)


def _tc2_body(o_ref, b_ref, ws1_ref, bs1_ref, ws2_ref, bs2_ref,
              gs_ref, gd_ref, w2bc_ref, b2bc_ref):
    x2 = o_ref[0] + o_ref[1] + b_ref[...]
    gs_ref[...] = jnp.dot(x2, ws1_ref[0:H, :], precision=_HIGH)
    gd_ref[...] = jnp.dot(x2, ws1_ref[H:2 * H, :], precision=_HIGH) + bs1_ref[...]
    w2bc_ref[...] = jnp.broadcast_to(ws2_ref[...], (H, 16))
    b2bc_ref[...] = jnp.broadcast_to(bs2_ref[...], (1, 16))


def _tcm_body(dn_ref, dnm_ref):
    dnm_ref[...] = dn_ref[0] + dn_ref[1]


_tcm = pl.pallas_call(
    _tcm_body,
    out_shape=jax.ShapeDtypeStruct((NP,), f32),
)


_tc2 = pl.pallas_call(
    _tc2_body,
    out_shape=(
        jax.ShapeDtypeStruct((NP, H), f32),
        jax.ShapeDtypeStruct((NP, H), f32),
        jax.ShapeDtypeStruct((H, 16), f32),
        jax.ShapeDtypeStruct((1, 16), f32),
    ),
)


@functools.partial(
    pl.kernel,
    out_type=(
        jax.ShapeDtypeStruct((E,), f32),
        jax.ShapeDtypeStruct((2, NP), f32),
    ),
    mesh=_mesh,
    compiler_params=_sc_params,
    scratch_types=[
        pltpu.VMEM((2 * NP,), f32),
        pltpu.VMEM((DE, 16), f32),
        pltpu.VMEM((1, 16), f32),
        pltpu.VMEM((C,), i32),
        pltpu.VMEM((C,), i32),
        pltpu.VMEM((C,), i32),
        pltpu.VMEM((C,), i32),
        pltpu.VMEM((SAB * C,), f32),
        pltpu.VMEM((C, DE), f32),
        pltpu.VMEM((C, DE), f32),
        pltpu.VMEM((C,), f32),
        pltpu.VMEM((NP,), f32),
        pltpu.VMEM_SHARED((16, NP), f32),
        pltpu.VMEM((16, RPT), f32),
        pltpu.VMEM((RPT,), f32),
        pltpu.SemaphoreType.DMA,
        pltpu.SemaphoreType.DMA,
    ],
)
def _pass_a(ei_hbm, ea_hbm, sd_hbm, ubc_hbm, cb_hbm, p_hbm, dn_hbm,
            sd_v, ubc_v, cb_v, src0, src1, dst0, dst1, p_v, ea0, ea1, xp,
            dloc, dsh, red, outb, sem0, sem1):
    c = lax.axis_index("c")
    s = lax.axis_index("s")
    w = s * 2 + c
    span0 = w * SAB
    eoff0 = span0 * C
    pltpu.sync_copy(sd_hbm, sd_v)
    pltpu.sync_copy(ubc_hbm, ubc_v)
    pltpu.sync_copy(cb_hbm, cb_v)
    cbv = cb_v[0]
    uvs = [ubc_v[k] for k in range(DE)]
    iota = lax.iota(i32, 16)
    zeros16 = jnp.zeros((16,), f32)
    ones16i = jnp.full((16,), 1, i32)
    srcb = [src0, src1]
    dstb = [dst0, dst1]
    eabuf = [ea0, ea1]
    sems = [sem0, sem1]

    @pl.loop(0, NP, step=16)
    def _(i):
        dloc[pl.ds(i, 16)] = zeros16

    def fire(slot, it):
        off = (span0 + it) * C
        pltpu.async_copy(ei_hbm.at[0, pl.ds(off, C)], srcb[slot], sems[slot])
        pltpu.async_copy(ei_hbm.at[1, pl.ds(off, C)], dstb[slot], sems[slot])
        pltpu.async_copy(ea_hbm.at[pl.ds(off, C), :], eabuf[slot], sems[slot])

    def wait(slot, it):
        off = (span0 + it) * C
        pltpu.make_async_copy(ei_hbm.at[0, pl.ds(off, C)], srcb[slot],
                              sems[slot]).wait()
        pltpu.make_async_copy(ei_hbm.at[1, pl.ds(off, C)], dstb[slot],
                              sems[slot]).wait()
        pltpu.make_async_copy(ea_hbm.at[pl.ds(off, C), :], eabuf[slot],
                              sems[slot]).wait()

    def compute(slot, pv_ref, pbase):
        sv_ref = srcb[slot]
        dv_ref = dstb[slot]
        ea_v = eabuf[slot]

        @pl.loop(0, C, step=16)
        def _(g):
            sv = sv_ref[pl.ds(g, 16)]
            dv = dv_ref[pl.ds(g, 16)]
            ev = iota + g
            acc = cbv
            for k in range(DE):
                kc = jnp.full((16,), k, i32)
                acc = acc + plsc.load_gather(ea_v, [ev, kc]) * uvs[k]
            a_s = plsc.load_gather(sd_v, [sv + sv])
            a_d = plsc.load_gather(sd_v, [dv + dv + ones16i])
            lg = a_s + a_d + acc
            lg = jnp.maximum(lg, lg * 0.2)
            pe = jnp.exp(lg)
            pv_ref[pl.ds(pbase + g, 16)] = pe
            plsc.addupdate_scatter(dloc, [dv], pe)

    fire(0, 0)
    fire(1, 1)

    @pl.loop(0, SAB - 1, step=2)
    def _(it0):
        wait(0, it0)
        compute(0, p_v, it0 * C)
        fire(0, it0 + 2)
        wait(1, it0 + 1)
        compute(1, p_v, (it0 + 1) * C)

        @pl.when(it0 + 3 < SAB)
        def _():
            fire(1, it0 + 3)

    wait(0, SAB - 1)
    compute(0, p_v, (SAB - 1) * C)
    pltpu.sync_copy(p_v, p_hbm.at[pl.ds(eoff0, SAB * C)])

    @pl.when(w < XAB)
    def _():
        xoff = (NWK * SAB + w) * C
        pltpu.sync_copy(ei_hbm.at[0, pl.ds(xoff, C)], src0)
        pltpu.sync_copy(ei_hbm.at[1, pl.ds(xoff, C)], dst0)
        pltpu.sync_copy(ea_hbm.at[pl.ds(xoff, C), :], ea0)
        compute(0, xp, 0)
        pltpu.sync_copy(xp, p_hbm.at[pl.ds(xoff, C)])

    pltpu.sync_copy(dloc, dsh.at[s])
    plsc.subcore_barrier()
    pltpu.sync_copy(dsh.at[:, pl.ds(s * RPT, RPT)], red)
    for j in range(0, RPT, 16):
        acc = red[0, pl.ds(j, 16)]
        for t in range(1, 16):
            acc = acc + red[t, pl.ds(j, 16)]
        outb[pl.ds(j, 16)] = acc
    pltpu.sync_copy(outb, dn_hbm.at[c, pl.ds(s * RPT, RPT)])


@functools.partial(
    pl.kernel,
    out_type=jax.ShapeDtypeStruct((2, NP, H), f32),
    mesh=_mesh,
    compiler_params=_sc_params,
    scratch_types=[
        pltpu.VMEM((NP,), f32),
        pltpu.VMEM((SC_ * 2, 128), i32),
        pltpu.VMEM((2, 128), i32),
        pltpu.VMEM((2, 128), i32),
        pltpu.VMEM((CC,), f32),
        pltpu.VMEM((CC,), f32),
        pltpu.VMEM((CC, H), f32),
        pltpu.VMEM((CC, H), f32),
        pltpu.VMEM_SHARED((NP, H), f32),
        pltpu.SemaphoreType.DMA,
        pltpu.SemaphoreType.DMA,
    ],
)
def _pass_b(eir_hbm, p_hbm, dn_hbm, h_hbm, out_hbm,
            dtab, src2, d20, d21, pb0, pb1, rb0, rb1,
            osh, sem0, sem1):
    c = lax.axis_index("c")
    s = lax.axis_index("s")
    w = s * 2 + c
    span0 = w * SC_
    iota = lax.iota(i32, 16)
    zeros16 = jnp.zeros((16,), f32)
    d2b = [d20, d21]
    pbb = [pb0, pb1]
    rbb = [rb0, rb1]
    sems = [sem0, sem1]

    pltpu.sync_copy(dn_hbm, dtab)

    @pl.loop(0, CC)
    def _(r):
        for kk in range(H // 16):
            rb0[r, pl.ds(kk * 16, 16)] = zeros16

    pltpu.sync_copy(rb0.at[pl.ds(0, 256), :], osh.at[pl.ds(s * RPT, 256), :])
    pltpu.sync_copy(rb0.at[pl.ds(0, 256), :],
                    osh.at[pl.ds(s * RPT + 256, 256), :])
    pltpu.sync_copy(rb0.at[pl.ds(0, 128), :],
                    osh.at[pl.ds(s * RPT + 512, 128), :])
    plsc.subcore_barrier()

    pltpu.sync_copy(eir_hbm.at[0, pl.ds(span0 * 2, SC_ * 2), :], src2)

    def fire(slot, it):
        off = (span0 + it) * CC
        pltpu.async_copy(eir_hbm.at[1, pl.ds((span0 + it) * 2, 2), :],
                         d2b[slot], sems[slot])
        pltpu.async_copy(p_hbm.at[pl.ds(off, CC)], pbb[slot], sems[slot])
        for j in range(2):
            pltpu.async_copy(h_hbm.at[src2.at[it * 2 + j]],
                             rbb[slot].at[pl.ds(j * 128, 128), :], sems[slot])

    def wait(slot, it):
        off = (span0 + it) * CC
        pltpu.make_async_copy(eir_hbm.at[1, pl.ds((span0 + it) * 2, 2), :],
                              d2b[slot], sems[slot]).wait()
        pltpu.make_async_copy(p_hbm.at[pl.ds(off, CC)], pbb[slot],
                              sems[slot]).wait()
        for j in range(2):
            pltpu.make_async_copy(h_hbm.at[src2.at[it * 2 + j]],
                                  rbb[slot].at[pl.ds(j * 128, 128), :],
                                  sems[slot]).wait()

    def compute(slot):
        d2_ref = d2b[slot]
        p_ref = pbb[slot]
        rb_ref = rbb[slot]

        for r in range(2):
            @pl.loop(0, 128, step=16)
            def _(g):
                dv = d2_ref[r, pl.ds(g, 16)]
                pe = p_ref[pl.ds(r * 128 + g, 16)]
                dn = plsc.load_gather(dtab, [dv])
                cf = pe / (dn + 1e-16)
                ev = iota + r * 128 + g
                for k in range(H):
                    kc = jnp.full((16,), k, i32)
                    v = plsc.load_gather(rb_ref, [ev, kc])
                    plsc.store_scatter(rb_ref, [ev, kc], v * cf)

    def scatter(slot):
        for j in range(2):
            pltpu.sync_copy(rbb[slot].at[pl.ds(j * 128, 128), :],
                            osh.at[d2b[slot].at[j]], add=True)

    fire(0, 0)
    fire(1, 1)

    @pl.loop(0, SC_ - 1, step=2)
    def _(it0):
        wait(0, it0)
        compute(0)
        scatter(0)
        fire(0, it0 + 2)
        wait(1, it0 + 1)
        compute(1)
        scatter(1)

        @pl.when(it0 + 3 < SC_)
        def _():
            fire(1, it0 + 3)

    wait(0, SC_ - 1)
    compute(0)
    scatter(0)

    @pl.when(w < XC)
    def _():
        xoff = (NWK * SC_ + w) * CC
        pltpu.sync_copy(eir_hbm.at[0, pl.ds((NWK * SC_ + w) * 2, 2), :], d20)
        pltpu.sync_copy(eir_hbm.at[1, pl.ds((NWK * SC_ + w) * 2, 2), :], d21)
        pltpu.sync_copy(p_hbm.at[pl.ds(xoff, CC)], pb1)
        for j in range(2):
            pltpu.sync_copy(h_hbm.at[d20.at[j]],
                            rb1.at[pl.ds(j * 128, 128), :])
        compute(1)
        scatter(1)

    plsc.subcore_barrier()
    for q, nrow in ((0, 256), (256, 256), (512, 128)):
        pltpu.sync_copy(osh.at[pl.ds(s * RPT + q, nrow), :],
                        rb0.at[pl.ds(0, nrow), :])
        pltpu.sync_copy(rb0.at[pl.ds(0, nrow), :],
                        out_hbm.at[c, pl.ds(s * RPT + q, nrow), :])


@functools.partial(
    pl.kernel,
    out_type=jax.ShapeDtypeStruct((E,), f32),
    mesh=_mesh,
    compiler_params=_sc_params,
    scratch_types=[
        pltpu.VMEM((SC_ * 2, 128), i32),
        pltpu.VMEM((SC_ * 2, 128), i32),
        pltpu.VMEM((CC, H), f32),
        pltpu.VMEM((CC, H), f32),
        pltpu.VMEM((CC, H), f32),
        pltpu.VMEM((CC, H), f32),
        pltpu.VMEM((H, 16), f32),
        pltpu.VMEM((1, 16), f32),
        pltpu.VMEM((SC_ * CC,), f32),
        pltpu.VMEM((2, 128), i32),
        pltpu.VMEM((2, 128), i32),
        pltpu.VMEM((CC,), f32),
        pltpu.SemaphoreType.DMA,
        pltpu.SemaphoreType.DMA,
    ],
)
def _pass_c(eir_hbm, gs_hbm, gd_hbm, w2bc_hbm, b2bc_hbm, sc_hbm,
            src2, dst2, sb0, sb1, db0, db1, w2_v, b2_v, sc_v,
            xsrc2, xdst2, xsc, sem0, sem1):
    c = lax.axis_index("c")
    s = lax.axis_index("s")
    w = s * 2 + c
    span0 = w * SC_
    iota = lax.iota(i32, 16)
    sbb = [sb0, sb1]
    dbb = [db0, db1]
    sems = [sem0, sem1]
    pltpu.sync_copy(w2bc_hbm, w2_v)
    pltpu.sync_copy(b2bc_hbm, b2_v)
    pltpu.sync_copy(eir_hbm.at[0, pl.ds(span0 * 2, SC_ * 2), :], src2)
    pltpu.sync_copy(eir_hbm.at[1, pl.ds(span0 * 2, SC_ * 2), :], dst2)
    bias = b2_v[0]

    def fire(slot, it):
        for j in range(2):
            pltpu.async_copy(gs_hbm.at[src2.at[it * 2 + j]],
                             sbb[slot].at[pl.ds(j * 128, 128), :], sems[slot])
            pltpu.async_copy(gd_hbm.at[dst2.at[it * 2 + j]],
                             dbb[slot].at[pl.ds(j * 128, 128), :], sems[slot])

    def wait(slot, it):
        for j in range(2):
            pltpu.make_async_copy(gs_hbm.at[src2.at[it * 2 + j]],
                                  sbb[slot].at[pl.ds(j * 128, 128), :],
                                  sems[slot]).wait()
            pltpu.make_async_copy(gd_hbm.at[dst2.at[it * 2 + j]],
                                  dbb[slot].at[pl.ds(j * 128, 128), :],
                                  sems[slot]).wait()

    def compute(slot, out_ref, base):
        sbuf = sbb[slot]
        dbuf = dbb[slot]

        @pl.loop(0, CC, step=16)
        def _(g):
            ev = iota + g
            acc = bias
            for k in range(H):
                kc = jnp.full((16,), k, i32)
                t = (plsc.load_gather(sbuf, [ev, kc])
                     + plsc.load_gather(dbuf, [ev, kc]))
                acc = acc + jnp.maximum(t, 0.0) * w2_v[k]
            out_ref[pl.ds(base + g, 16)] = 1.0 / (1.0 + jnp.exp(-acc))

    fire(0, 0)
    fire(1, 1)

    @pl.loop(0, SC_ - 1, step=2)
    def _(it0):
        wait(0, it0)
        compute(0, sc_v, it0 * CC)
        fire(0, it0 + 2)
        wait(1, it0 + 1)
        compute(1, sc_v, (it0 + 1) * CC)

        @pl.when(it0 + 3 < SC_)
        def _():
            fire(1, it0 + 3)

    wait(0, SC_ - 1)
    compute(0, sc_v, (SC_ - 1) * CC)
    pltpu.sync_copy(sc_v, sc_hbm.at[pl.ds(span0 * CC, SC_ * CC)])

    @pl.when(w < XC)
    def _():
        xoff = (NWK * SC_ + w) * CC
        pltpu.sync_copy(eir_hbm.at[0, pl.ds((NWK * SC_ + w) * 2, 2), :], xsrc2)
        pltpu.sync_copy(eir_hbm.at[1, pl.ds((NWK * SC_ + w) * 2, 2), :], xdst2)
        for j in range(2):
            pltpu.sync_copy(gs_hbm.at[xsrc2.at[j]],
                            sb0.at[pl.ds(j * 128, 128), :])
            pltpu.sync_copy(gd_hbm.at[xdst2.at[j]],
                            db0.at[pl.ds(j * 128, 128), :])
        compute(0, xsc, 0)
        pltpu.sync_copy(xsc, sc_hbm.at[pl.ds(xoff, CC)])


def kernel(x, edge_index, edge_attr, We, be, W1, We1, a_s1, a_d1, a_e1, b1,
           W2, We2, a_s2, a_d2, a_e2, b2, Ws1, bs1, Ws2, bs2):
    x_p = jnp.pad(x, ((0, NP - N), (0, 0)))
    A1 = jnp.stack([a_s1, a_d1], axis=1)
    A2 = jnp.stack([a_s2, a_d2], axis=1)
    eir = edge_index.reshape(2, ER, 128)

    h1, sd1, ubc1, ubc2, cbc = _tc0(
        x_p, W1, A1, We, be[None, :], We1, a_e1[:, None], We2, a_e2[:, None])

    p1, dn1 = _pass_a(edge_index, edge_attr, sd1.reshape(-1), ubc1, cbc[0:1])
    o1 = _pass_b(eir, p1, _tcm(dn1), h1)

    h2, sd2 = _tc1(o1, b1[None, :], W2, A2)

    p2, dn2 = _pass_a(edge_index, edge_attr, sd2.reshape(-1), ubc2, cbc[1:2])
    o2 = _pass_b(eir, p2, _tcm(dn2), h2)

    gs, gd, w2bc, b2bc = _tc2(
        o2, b2[None, :], Ws1, bs1[None, :], Ws2, bs2[:, None])

    return _pass_c(eir, gs, gd, w2bc, b2bc)


# speedup vs baseline: 9.0896x; 1.1270x over previous
import functools

import jax
import jax.numpy as jnp
from jax import lax
from jax.experimental import pallas as pl
from jax.experimental.pallas import tpu as pltpu
from jax.experimental.pallas import tpu_sc as plsc

N = 10000
E = 320000
D = 128
DE = 16
H = 64

NP = 10240
NWK = 32
RPT = NP // 16

C = 512
NCH = E // C
SAB = NCH // NWK
XAB = NCH - SAB * NWK

CC = 256
NCHC = E // CC
SC_ = NCHC // NWK
XC = NCHC - SC_ * NWK

ER = E // 128

f32 = jnp.float32
i32 = jnp.int32

_HIGH = lax.Precision.HIGHEST

_mesh = plsc.VectorSubcoreMesh(core_axis_name="c", subcore_axis_name="s")
_sc_params = pltpu.CompilerParams(needs_layout_passes=False,
                                  use_tc_tiling_on_sc=False)


def _tc0_body(x_ref, w1_ref, a1_ref, we_ref, be_ref, we1_ref, ae1_ref,
              we2_ref, ae2_ref, h_ref, sd_ref, ubc1_ref, ubc2_ref, cbc_ref):
    h = jnp.dot(x_ref[...], w1_ref[...], precision=_HIGH)
    h_ref[...] = h
    sd_ref[...] = jnp.dot(h, a1_ref[...], precision=_HIGH)
    v1 = jnp.dot(we1_ref[...], ae1_ref[...], precision=_HIGH)
    v2 = jnp.dot(we2_ref[...], ae2_ref[...], precision=_HIGH)
    u1 = jnp.dot(we_ref[...], v1, precision=_HIGH)
    u2 = jnp.dot(we_ref[...], v2, precision=_HIGH)
    ubc1_ref[...] = jnp.broadcast_to(u1, (DE, 16))
    ubc2_ref[...] = jnp.broadcast_to(u2, (DE, 16))
    c1 = jnp.dot(be_ref[...], v1, precision=_HIGH)
    c2 = jnp.dot(be_ref[...], v2, precision=_HIGH)
    cbc_ref[...] = jnp.concatenate(
        [jnp.broadcast_to(c1, (1, 16)), jnp.broadcast_to(c2, (1, 16))], axis=0)


_tc0 = pl.pallas_call(
    _tc0_body,
    out_shape=(
        jax.ShapeDtypeStruct((NP, H), f32),
        jax.ShapeDtypeStruct((NP, 2), f32),
        jax.ShapeDtypeStruct((DE, 16), f32),
        jax.ShapeDtypeStruct((DE, 16), f32),
        jax.ShapeDtypeStruct((2, 16), f32),
    ),
)


def _tc1_body(o_ref, b_ref, w2_ref, a2_ref, h2_ref, sd2_ref):
    x1 = jnp.maximum(o_ref[0] + o_ref[1] + b_ref[...], 0.0)
    h2 = jnp.dot(x1, w2_ref[...], precision=_HIGH)
    h2_ref[...] = h2
    sd2_ref[...] = jnp.dot(h2, a2_ref[...], precision=_HIGH)


_tc1 = pl.pallas_call(
    _tc1_body,
    out_shape=(
        jax.ShapeDtypeStruct((NP, H), f32),
        jax.ShapeDtypeStruct((NP, 2), f32),
    ),
)


def _tc2_body(o_ref, b_ref, ws1_ref, bs1_ref, ws2_ref, bs2_ref,
              gs_ref, gd_ref, w2bc_ref, b2bc_ref):
    x2 = o_ref[0] + o_ref[1] + b_ref[...]
    gs_ref[...] = jnp.dot(x2, ws1_ref[0:H, :], precision=_HIGH)
    gd_ref[...] = jnp.dot(x2, ws1_ref[H:2 * H, :], precision=_HIGH) + bs1_ref[...]
    w2bc_ref[...] = jnp.broadcast_to(ws2_ref[...], (H, 16))
    b2bc_ref[...] = jnp.broadcast_to(bs2_ref[...], (1, 16))


def _tcm_body(dn_ref, dnm_ref):
    dnm_ref[...] = dn_ref[0] + dn_ref[1]


_tcm = pl.pallas_call(
    _tcm_body,
    out_shape=jax.ShapeDtypeStruct((NP,), f32),
)


_tc2 = pl.pallas_call(
    _tc2_body,
    out_shape=(
        jax.ShapeDtypeStruct((NP, H), f32),
        jax.ShapeDtypeStruct((NP, H), f32),
        jax.ShapeDtypeStruct((H, 16), f32),
        jax.ShapeDtypeStruct((1, 16), f32),
    ),
)


@functools.partial(
    pl.kernel,
    out_type=(
        jax.ShapeDtypeStruct((E,), f32),
        jax.ShapeDtypeStruct((2, NP), f32),
    ),
    mesh=_mesh,
    compiler_params=_sc_params,
    scratch_types=[
        pltpu.VMEM((2 * NP,), f32),
        pltpu.VMEM((DE, 16), f32),
        pltpu.VMEM((1, 16), f32),
        pltpu.VMEM((C,), i32),
        pltpu.VMEM((C,), i32),
        pltpu.VMEM((C,), i32),
        pltpu.VMEM((C,), i32),
        pltpu.VMEM((SAB * C,), f32),
        pltpu.VMEM((C, DE), f32),
        pltpu.VMEM((C, DE), f32),
        pltpu.VMEM((C,), f32),
        pltpu.VMEM((NP,), f32),
        pltpu.VMEM_SHARED((16, NP), f32),
        pltpu.VMEM((16, RPT), f32),
        pltpu.VMEM((RPT,), f32),
        pltpu.SemaphoreType.DMA,
        pltpu.SemaphoreType.DMA,
    ],
)
def _pass_a(ei_hbm, ea_hbm, sd_hbm, ubc_hbm, cb_hbm, p_hbm, dn_hbm,
            sd_v, ubc_v, cb_v, src0, src1, dst0, dst1, p_v, ea0, ea1, xp,
            dloc, dsh, red, outb, sem0, sem1):
    c = lax.axis_index("c")
    s = lax.axis_index("s")
    w = s * 2 + c
    span0 = w * SAB
    eoff0 = span0 * C
    pltpu.sync_copy(sd_hbm, sd_v)
    pltpu.sync_copy(ubc_hbm, ubc_v)
    pltpu.sync_copy(cb_hbm, cb_v)
    cbv = cb_v[0]
    uvs = [ubc_v[k] for k in range(DE)]
    iota = lax.iota(i32, 16)
    zeros16 = jnp.zeros((16,), f32)
    ones16i = jnp.full((16,), 1, i32)
    srcb = [src0, src1]
    dstb = [dst0, dst1]
    eabuf = [ea0, ea1]
    sems = [sem0, sem1]

    @pl.loop(0, NP, step=16)
    def _(i):
        dloc[pl.ds(i, 16)] = zeros16

    def fire(slot, it):
        off = (span0 + it) * C
        pltpu.async_copy(ei_hbm.at[0, pl.ds(off, C)], srcb[slot], sems[slot])
        pltpu.async_copy(ei_hbm.at[1, pl.ds(off, C)], dstb[slot], sems[slot])
        pltpu.async_copy(ea_hbm.at[pl.ds(off, C), :], eabuf[slot], sems[slot])

    def wait(slot, it):
        off = (span0 + it) * C
        pltpu.make_async_copy(ei_hbm.at[0, pl.ds(off, C)], srcb[slot],
                              sems[slot]).wait()
        pltpu.make_async_copy(ei_hbm.at[1, pl.ds(off, C)], dstb[slot],
                              sems[slot]).wait()
        pltpu.make_async_copy(ea_hbm.at[pl.ds(off, C), :], eabuf[slot],
                              sems[slot]).wait()

    def compute(slot, pv_ref, pbase):
        sv_ref = srcb[slot]
        dv_ref = dstb[slot]
        ea_v = eabuf[slot]

        @pl.loop(0, C, step=16)
        def _(g):
            sv = sv_ref[pl.ds(g, 16)]
            dv = dv_ref[pl.ds(g, 16)]
            ev = iota + g
            acc = cbv
            for k in range(DE):
                kc = jnp.full((16,), k, i32)
                acc = acc + plsc.load_gather(ea_v, [ev, kc]) * uvs[k]
            a_s = plsc.load_gather(sd_v, [sv + sv])
            a_d = plsc.load_gather(sd_v, [dv + dv + ones16i])
            lg = a_s + a_d + acc
            lg = jnp.maximum(lg, lg * 0.2)
            pe = jnp.exp(lg)
            pv_ref[pl.ds(pbase + g, 16)] = pe
            plsc.addupdate_scatter(dloc, [dv], pe)

    fire(0, 0)
    fire(1, 1)

    @pl.loop(0, SAB - 1, step=2)
    def _(it0):
        wait(0, it0)
        compute(0, p_v, it0 * C)
        fire(0, it0 + 2)
        wait(1, it0 + 1)
        compute(1, p_v, (it0 + 1) * C)

        @pl.when(it0 + 3 < SAB)
        def _():
            fire(1, it0 + 3)

    wait(0, SAB - 1)
    compute(0, p_v, (SAB - 1) * C)
    pltpu.sync_copy(p_v, p_hbm.at[pl.ds(eoff0, SAB * C)])

    @pl.when(w < XAB)
    def _():
        xoff = (NWK * SAB + w) * C
        pltpu.sync_copy(ei_hbm.at[0, pl.ds(xoff, C)], src0)
        pltpu.sync_copy(ei_hbm.at[1, pl.ds(xoff, C)], dst0)
        pltpu.sync_copy(ea_hbm.at[pl.ds(xoff, C), :], ea0)
        compute(0, xp, 0)
        pltpu.sync_copy(xp, p_hbm.at[pl.ds(xoff, C)])

    pltpu.sync_copy(dloc, dsh.at[s])
    plsc.subcore_barrier()
    pltpu.sync_copy(dsh.at[:, pl.ds(s * RPT, RPT)], red)
    for j in range(0, RPT, 16):
        acc = red[0, pl.ds(j, 16)]
        for t in range(1, 16):
            acc = acc + red[t, pl.ds(j, 16)]
        outb[pl.ds(j, 16)] = acc
    pltpu.sync_copy(outb, dn_hbm.at[c, pl.ds(s * RPT, RPT)])


@functools.partial(
    pl.kernel,
    out_type=jax.ShapeDtypeStruct((2, NP, H), f32),
    mesh=_mesh,
    compiler_params=_sc_params,
    scratch_types=[
        pltpu.VMEM((NP,), f32),
        pltpu.VMEM((SC_ * 2, 128), i32),
        pltpu.VMEM((2, 128), i32),
        pltpu.VMEM((2, 128), i32),
        pltpu.VMEM((CC,), f32),
        pltpu.VMEM((CC,), f32),
        pltpu.VMEM((CC, H), f32),
        pltpu.VMEM((CC, H), f32),
        pltpu.VMEM_SHARED((NP, H), f32),
        pltpu.SemaphoreType.DMA,
        pltpu.SemaphoreType.DMA,
        pltpu.SemaphoreType.DMA,
        pltpu.SemaphoreType.DMA,
    ],
)
def _pass_b(eir_hbm, p_hbm, dn_hbm, h_hbm, out_hbm,
            dtab, src2, d20, d21, pb0, pb1, rb0, rb1,
            osh, sem0, sem1, tsem0, tsem1):
    c = lax.axis_index("c")
    s = lax.axis_index("s")
    w = s * 2 + c
    span0 = w * SC_
    iota = lax.iota(i32, 16)
    zeros16 = jnp.zeros((16,), f32)
    d2b = [d20, d21]
    pbb = [pb0, pb1]
    rbb = [rb0, rb1]
    sems = [sem0, sem1]
    tsems = [tsem0, tsem1]

    pltpu.sync_copy(dn_hbm, dtab)

    @pl.loop(0, CC)
    def _(r):
        for kk in range(H // 16):
            rb0[r, pl.ds(kk * 16, 16)] = zeros16

    pltpu.sync_copy(rb0.at[pl.ds(0, 256), :], osh.at[pl.ds(s * RPT, 256), :])
    pltpu.sync_copy(rb0.at[pl.ds(0, 256), :],
                    osh.at[pl.ds(s * RPT + 256, 256), :])
    pltpu.sync_copy(rb0.at[pl.ds(0, 128), :],
                    osh.at[pl.ds(s * RPT + 512, 128), :])
    plsc.subcore_barrier()

    pltpu.sync_copy(eir_hbm.at[0, pl.ds(span0 * 2, SC_ * 2), :], src2)

    def fire(slot, it):
        off = (span0 + it) * CC
        pltpu.async_copy(eir_hbm.at[1, pl.ds((span0 + it) * 2, 2), :],
                         d2b[slot], sems[slot])
        pltpu.async_copy(p_hbm.at[pl.ds(off, CC)], pbb[slot], sems[slot])
        for j in range(2):
            pltpu.async_copy(h_hbm.at[src2.at[it * 2 + j]],
                             rbb[slot].at[pl.ds(j * 128, 128), :], sems[slot])

    def wait(slot, it):
        off = (span0 + it) * CC
        pltpu.make_async_copy(eir_hbm.at[1, pl.ds((span0 + it) * 2, 2), :],
                              d2b[slot], sems[slot]).wait()
        pltpu.make_async_copy(p_hbm.at[pl.ds(off, CC)], pbb[slot],
                              sems[slot]).wait()
        for j in range(2):
            pltpu.make_async_copy(h_hbm.at[src2.at[it * 2 + j]],
                                  rbb[slot].at[pl.ds(j * 128, 128), :],
                                  sems[slot]).wait()

    def compute(slot):
        d2_ref = d2b[slot]
        p_ref = pbb[slot]
        rb_ref = rbb[slot]

        for r in range(2):
            @pl.loop(0, 128, step=16)
            def _(g):
                dv = d2_ref[r, pl.ds(g, 16)]
                pe = p_ref[pl.ds(r * 128 + g, 16)]
                dn = plsc.load_gather(dtab, [dv])
                cf = pe / (dn + 1e-16)
                ev = iota + r * 128 + g
                for k in range(H):
                    kc = jnp.full((16,), k, i32)
                    v = plsc.load_gather(rb_ref, [ev, kc])
                    plsc.store_scatter(rb_ref, [ev, kc], v * cf)

    def scatter_fire(slot):
        for j in range(2):
            pltpu.async_copy(rbb[slot].at[pl.ds(j * 128, 128), :],
                             osh.at[d2b[slot].at[j]], tsems[slot], add=True)

    def scatter_wait(slot):
        for j in range(2):
            pltpu.make_async_copy(rbb[slot].at[pl.ds(j * 128, 128), :],
                                  osh.at[d2b[slot].at[j]],
                                  tsems[slot]).wait()

    fire(0, 0)
    fire(1, 1)

    @pl.loop(0, SC_ - 1, step=2)
    def _(it0):
        wait(0, it0)
        compute(0)
        scatter_fire(0)
        wait(1, it0 + 1)
        compute(1)
        scatter_fire(1)
        scatter_wait(0)
        fire(0, it0 + 2)
        scatter_wait(1)

        @pl.when(it0 + 3 < SC_)
        def _():
            fire(1, it0 + 3)

    wait(0, SC_ - 1)
    compute(0)
    scatter_fire(0)
    scatter_wait(0)

    @pl.when(w < XC)
    def _():
        xoff = (NWK * SC_ + w) * CC
        pltpu.sync_copy(eir_hbm.at[0, pl.ds((NWK * SC_ + w) * 2, 2), :], d20)
        pltpu.sync_copy(eir_hbm.at[1, pl.ds((NWK * SC_ + w) * 2, 2), :], d21)
        pltpu.sync_copy(p_hbm.at[pl.ds(xoff, CC)], pb1)
        for j in range(2):
            pltpu.sync_copy(h_hbm.at[d20.at[j]],
                            rb1.at[pl.ds(j * 128, 128), :])
        compute(1)
        scatter_fire(1)
        scatter_wait(1)

    plsc.subcore_barrier()
    for q, nrow in ((0, 256), (256, 256), (512, 128)):
        pltpu.sync_copy(osh.at[pl.ds(s * RPT + q, nrow), :],
                        rb0.at[pl.ds(0, nrow), :])
        pltpu.sync_copy(rb0.at[pl.ds(0, nrow), :],
                        out_hbm.at[c, pl.ds(s * RPT + q, nrow), :])


@functools.partial(
    pl.kernel,
    out_type=jax.ShapeDtypeStruct((E,), f32),
    mesh=_mesh,
    compiler_params=_sc_params,
    scratch_types=[
        pltpu.VMEM((SC_ * 2, 128), i32),
        pltpu.VMEM((SC_ * 2, 128), i32),
        pltpu.VMEM((CC, H), f32),
        pltpu.VMEM((CC, H), f32),
        pltpu.VMEM((H, 16), f32),
        pltpu.VMEM((1, 16), f32),
        pltpu.VMEM((SC_ * CC,), f32),
        pltpu.VMEM((2, 128), i32),
        pltpu.VMEM((2, 128), i32),
        pltpu.VMEM((CC,), f32),
        pltpu.SemaphoreType.DMA,
        pltpu.SemaphoreType.DMA,
        pltpu.SemaphoreType.DMA,
        pltpu.SemaphoreType.DMA,
    ],
)
def _pass_c(eir_hbm, gs_hbm, gd_hbm, w2bc_hbm, b2bc_hbm, sc_hbm,
            src2, dst2, fb0, fb1, w2_v, b2_v, sc_v,
            xsrc2, xdst2, xsc, sg0, sg1, sd0, sd1):
    c = lax.axis_index("c")
    s = lax.axis_index("s")
    w = s * 2 + c
    span0 = w * SC_
    iota = lax.iota(i32, 16)
    fbb = [fb0, fb1]
    sgb = [sg0, sg1]
    sdb = [sd0, sd1]
    pltpu.sync_copy(w2bc_hbm, w2_v)
    pltpu.sync_copy(b2bc_hbm, b2_v)
    pltpu.sync_copy(eir_hbm.at[0, pl.ds(span0 * 2, SC_ * 2), :], src2)
    pltpu.sync_copy(eir_hbm.at[1, pl.ds(span0 * 2, SC_ * 2), :], dst2)
    bias = b2_v[0]

    def fire_gs(slot, it):
        for j in range(2):
            pltpu.async_copy(gs_hbm.at[src2.at[it * 2 + j]],
                             fbb[slot].at[pl.ds(j * 128, 128), :], sgb[slot])

    def wait_gs(slot, it):
        for j in range(2):
            pltpu.make_async_copy(gs_hbm.at[src2.at[it * 2 + j]],
                                  fbb[slot].at[pl.ds(j * 128, 128), :],
                                  sgb[slot]).wait()

    def fire_gd(slot, it):
        for j in range(2):
            pltpu.async_copy(gd_hbm.at[dst2.at[it * 2 + j]],
                             fbb[slot].at[pl.ds(j * 128, 128), :], sdb[slot],
                             add=True)

    def wait_gd(slot, it):
        for j in range(2):
            pltpu.make_async_copy(gd_hbm.at[dst2.at[it * 2 + j]],
                                  fbb[slot].at[pl.ds(j * 128, 128), :],
                                  sdb[slot]).wait()

    def compute(slot, out_ref, base):
        fbuf = fbb[slot]

        @pl.loop(0, CC, step=16)
        def _(g):
            ev = iota + g
            acc = bias
            for k in range(H):
                kc = jnp.full((16,), k, i32)
                t = plsc.load_gather(fbuf, [ev, kc])
                acc = acc + jnp.maximum(t, 0.0) * w2_v[k]
            out_ref[pl.ds(base + g, 16)] = 1.0 / (1.0 + jnp.exp(-acc))

    fire_gs(0, 0)
    fire_gs(1, 1)
    wait_gs(0, 0)
    fire_gd(0, 0)

    @pl.loop(0, SC_ - 1, step=2)
    def _(it0):
        wait_gs(1, it0 + 1)
        fire_gd(1, it0 + 1)
        wait_gd(0, it0)
        compute(0, sc_v, it0 * CC)
        fire_gs(0, it0 + 2)
        wait_gd(1, it0 + 1)
        compute(1, sc_v, (it0 + 1) * CC)

        @pl.when(it0 + 3 < SC_)
        def _():
            fire_gs(1, it0 + 3)

        wait_gs(0, it0 + 2)
        fire_gd(0, it0 + 2)

    wait_gd(0, SC_ - 1)
    compute(0, sc_v, (SC_ - 1) * CC)
    pltpu.sync_copy(sc_v, sc_hbm.at[pl.ds(span0 * CC, SC_ * CC)])

    @pl.when(w < XC)
    def _():
        xoff = (NWK * SC_ + w) * CC
        pltpu.sync_copy(eir_hbm.at[0, pl.ds((NWK * SC_ + w) * 2, 2), :], xsrc2)
        pltpu.sync_copy(eir_hbm.at[1, pl.ds((NWK * SC_ + w) * 2, 2), :], xdst2)
        for j in range(2):
            pltpu.async_copy(gs_hbm.at[xsrc2.at[j]],
                             fb0.at[pl.ds(j * 128, 128), :], sg0)
            pltpu.make_async_copy(gs_hbm.at[xsrc2.at[j]],
                                  fb0.at[pl.ds(j * 128, 128), :], sg0).wait()
            pltpu.async_copy(gd_hbm.at[xdst2.at[j]],
                             fb0.at[pl.ds(j * 128, 128), :], sd0, add=True)
            pltpu.make_async_copy(gd_hbm.at[xdst2.at[j]],
                                  fb0.at[pl.ds(j * 128, 128), :], sd0).wait()
        compute(0, xsc, 0)
        pltpu.sync_copy(xsc, sc_hbm.at[pl.ds(xoff, CC)])


def kernel(x, edge_index, edge_attr, We, be, W1, We1, a_s1, a_d1, a_e1, b1,
           W2, We2, a_s2, a_d2, a_e2, b2, Ws1, bs1, Ws2, bs2):
    x_p = jnp.pad(x, ((0, NP - N), (0, 0)))
    A1 = jnp.stack([a_s1, a_d1], axis=1)
    A2 = jnp.stack([a_s2, a_d2], axis=1)
    eir = edge_index.reshape(2, ER, 128)

    h1, sd1, ubc1, ubc2, cbc = _tc0(
        x_p, W1, A1, We, be[None, :], We1, a_e1[:, None], We2, a_e2[:, None])

    p1, dn1 = _pass_a(edge_index, edge_attr, sd1.reshape(-1), ubc1, cbc[0:1])
    o1 = _pass_b(eir, p1, _tcm(dn1), h1)

    h2, sd2 = _tc1(o1, b1[None, :], W2, A2)

    p2, dn2 = _pass_a(edge_index, edge_attr, sd2.reshape(-1), ubc2, cbc[1:2])
    o2 = _pass_b(eir, p2, _tcm(dn2), h2)

    gs, gd, w2bc, b2bc = _tc2(
        o2, b2[None, :], Ws1, bs1[None, :], Ws2, bs2[:, None])

    return _pass_c(eir, gs, gd, w2bc, b2bc)


# speedup vs baseline: 9.1949x; 1.0116x over previous
import functools

import jax
import jax.numpy as jnp
from jax import lax
from jax.experimental import pallas as pl
from jax.experimental.pallas import tpu as pltpu
from jax.experimental.pallas import tpu_sc as plsc

N = 10000
E = 320000
D = 128
DE = 16
H = 64

NP = 10240
NWK = 32
RPT = NP // 16

CC = 256
NCHC = E // CC
SC_ = NCHC // NWK
XC = NCHC - SC_ * NWK

ER = E // 128

f32 = jnp.float32
i32 = jnp.int32

_HIGH = lax.Precision.HIGHEST

_mesh = plsc.VectorSubcoreMesh(core_axis_name="c", subcore_axis_name="s")
_sc_params = pltpu.CompilerParams(needs_layout_passes=False,
                                  use_tc_tiling_on_sc=False)


def _tc0_body(x_ref, w1_ref, a1_ref, we_ref, be_ref, we1_ref, ae1_ref,
              we2_ref, ae2_ref, h_ref, sd_ref, ubc1_ref, ubc2_ref, cbc_ref):
    h = jnp.dot(x_ref[...], w1_ref[...], precision=_HIGH)
    h_ref[...] = h
    sd_ref[...] = jnp.dot(h, a1_ref[...], precision=_HIGH)
    v1 = jnp.dot(we1_ref[...], ae1_ref[...], precision=_HIGH)
    v2 = jnp.dot(we2_ref[...], ae2_ref[...], precision=_HIGH)
    u1 = jnp.dot(we_ref[...], v1, precision=_HIGH)
    u2 = jnp.dot(we_ref[...], v2, precision=_HIGH)
    ubc1_ref[...] = jnp.broadcast_to(u1, (DE, 16))
    ubc2_ref[...] = jnp.broadcast_to(u2, (DE, 16))
    c1 = jnp.dot(be_ref[...], v1, precision=_HIGH)
    c2 = jnp.dot(be_ref[...], v2, precision=_HIGH)
    cbc_ref[...] = jnp.concatenate(
        [jnp.broadcast_to(c1, (1, 16)), jnp.broadcast_to(c2, (1, 16))], axis=0)


def _col(vrow):
    nb = NP // 128
    i1 = lax.broadcasted_iota(i32, (nb, 128, 128), 1)
    i2 = lax.broadcasted_iota(i32, (nb, 128, 128), 2)
    vb = lax.broadcast_in_dim(vrow, (nb, 128, 128), (0, 2))
    return jnp.sum(jnp.where(i1 == i2, vb, 0.0), axis=2, keepdims=True)


_tc0 = pl.pallas_call(
    _tc0_body,
    out_shape=(
        jax.ShapeDtypeStruct((NP, H), f32),
        jax.ShapeDtypeStruct((NP, 2), f32),
        jax.ShapeDtypeStruct((DE, 16), f32),
        jax.ShapeDtypeStruct((DE, 16), f32),
        jax.ShapeDtypeStruct((2, 16), f32),
    ),
)


def _tc1_body(o_ref, dn_ref, b_ref, w2_ref, a2_ref, h2_ref, sd2_ref):
    dnm = jnp.sum(dn_ref[...], axis=(0, 1))
    inv3 = _col(1.0 / (dnm + 1e-16))
    o3 = jnp.reshape(o_ref[0] + o_ref[1], (NP // 128, 128, H))
    x1 = jnp.maximum(jnp.reshape(o3 * inv3, (NP, H)) + b_ref[...], 0.0)
    h2 = jnp.dot(x1, w2_ref[...], precision=_HIGH)
    h2_ref[...] = h2
    sd2_ref[...] = jnp.dot(h2, a2_ref[...], precision=_HIGH)


_tc1 = pl.pallas_call(
    _tc1_body,
    out_shape=(
        jax.ShapeDtypeStruct((NP, H), f32),
        jax.ShapeDtypeStruct((NP, 2), f32),
    ),
)


def _tc2_body(o_ref, dn_ref, b_ref, ws1_ref, bs1_ref, ws2_ref, bs2_ref,
              gs_ref, gd_ref, w2bc_ref, b2bc_ref):
    dnm = jnp.sum(dn_ref[...], axis=(0, 1))
    inv3 = _col(1.0 / (dnm + 1e-16))
    o3 = jnp.reshape(o_ref[0] + o_ref[1], (NP // 128, 128, H))
    x2 = jnp.reshape(o3 * inv3, (NP, H)) + b_ref[...]
    gs_ref[...] = jnp.dot(x2, ws1_ref[0:H, :], precision=_HIGH)
    gd_ref[...] = jnp.dot(x2, ws1_ref[H:2 * H, :], precision=_HIGH) + bs1_ref[...]
    w2bc_ref[...] = jnp.broadcast_to(ws2_ref[...], (H, 16))
    b2bc_ref[...] = jnp.broadcast_to(bs2_ref[...], (1, 16))


_tc2 = pl.pallas_call(
    _tc2_body,
    out_shape=(
        jax.ShapeDtypeStruct((NP, H), f32),
        jax.ShapeDtypeStruct((NP, H), f32),
        jax.ShapeDtypeStruct((H, 16), f32),
        jax.ShapeDtypeStruct((1, 16), f32),
    ),
)


@functools.partial(
    pl.kernel,
    out_type=(
        jax.ShapeDtypeStruct((2, NP, H), f32),
        jax.ShapeDtypeStruct((2, 16, NP), f32),
    ),
    mesh=_mesh,
    compiler_params=_sc_params,
    scratch_types=[
        pltpu.VMEM((2 * NP,), f32),
        pltpu.VMEM((DE, 16), f32),
        pltpu.VMEM((1, 16), f32),
        pltpu.VMEM((SC_ * 2, 128), i32),
        pltpu.VMEM((2, 128), i32),
        pltpu.VMEM((2, 128), i32),
        pltpu.VMEM((2, 128), i32),
        pltpu.VMEM((2, 128), i32),
        pltpu.VMEM((CC, DE), f32),
        pltpu.VMEM((CC, DE), f32),
        pltpu.VMEM((CC, H), f32),
        pltpu.VMEM((CC, H), f32),
        pltpu.VMEM((NP,), f32),
        pltpu.VMEM_SHARED((NP, H), f32),
        pltpu.SemaphoreType.DMA,
        pltpu.SemaphoreType.DMA,
        pltpu.SemaphoreType.DMA,
        pltpu.SemaphoreType.DMA,
    ],
)
def _pass_ab(eir_hbm, ea_hbm, sd_hbm, ubc_hbm, cb_hbm, h_hbm, out_hbm, dn_hbm,
             sd_v, ubc_v, cb_v, src2, s20, s21, d20, d21, ea0, ea1, rb0, rb1,
             dloc, osh, sem0, sem1, tsem0, tsem1):
    c = lax.axis_index("c")
    s = lax.axis_index("s")
    w = s * 2 + c
    span0 = w * SC_
    iota = lax.iota(i32, 16)
    zeros16 = jnp.zeros((16,), f32)
    ones16i = jnp.full((16,), 1, i32)
    s2b = [s20, s21]
    d2b = [d20, d21]
    eab = [ea0, ea1]
    rbb = [rb0, rb1]
    sems = [sem0, sem1]
    tsems = [tsem0, tsem1]

    pltpu.sync_copy(sd_hbm, sd_v)
    pltpu.sync_copy(ubc_hbm, ubc_v)
    pltpu.sync_copy(cb_hbm, cb_v)
    cbv = cb_v[0]
    uvs = [ubc_v[k] for k in range(DE)]

    @pl.loop(0, NP, step=16)
    def _(i):
        dloc[pl.ds(i, 16)] = zeros16

    @pl.loop(0, CC)
    def _(r):
        for kk in range(H // 16):
            rb0[r, pl.ds(kk * 16, 16)] = zeros16

    pltpu.sync_copy(rb0.at[pl.ds(0, 256), :], osh.at[pl.ds(s * RPT, 256), :])
    pltpu.sync_copy(rb0.at[pl.ds(0, 256), :],
                    osh.at[pl.ds(s * RPT + 256, 256), :])
    pltpu.sync_copy(rb0.at[pl.ds(0, 128), :],
                    osh.at[pl.ds(s * RPT + 512, 128), :])
    plsc.subcore_barrier()

    pltpu.sync_copy(eir_hbm.at[0, pl.ds(span0 * 2, SC_ * 2), :], src2)

    def fire(slot, it):
        off = (span0 + it) * CC
        pltpu.async_copy(eir_hbm.at[0, pl.ds((span0 + it) * 2, 2), :],
                         s2b[slot], sems[slot])
        pltpu.async_copy(eir_hbm.at[1, pl.ds((span0 + it) * 2, 2), :],
                         d2b[slot], sems[slot])
        pltpu.async_copy(ea_hbm.at[pl.ds(off, CC), :], eab[slot], sems[slot])
        for j in range(2):
            pltpu.async_copy(h_hbm.at[src2.at[it * 2 + j]],
                             rbb[slot].at[pl.ds(j * 128, 128), :], sems[slot])

    def wait(slot, it):
        off = (span0 + it) * CC
        pltpu.make_async_copy(eir_hbm.at[0, pl.ds((span0 + it) * 2, 2), :],
                              s2b[slot], sems[slot]).wait()
        pltpu.make_async_copy(eir_hbm.at[1, pl.ds((span0 + it) * 2, 2), :],
                              d2b[slot], sems[slot]).wait()
        pltpu.make_async_copy(ea_hbm.at[pl.ds(off, CC), :], eab[slot],
                              sems[slot]).wait()
        for j in range(2):
            pltpu.make_async_copy(h_hbm.at[src2.at[it * 2 + j]],
                                  rbb[slot].at[pl.ds(j * 128, 128), :],
                                  sems[slot]).wait()

    def compute(slot):
        s2_ref = s2b[slot]
        d2_ref = d2b[slot]
        ea_v = eab[slot]
        rb_ref = rbb[slot]

        for r in range(2):
            @pl.loop(0, 128, step=16)
            def _(g):
                sv = s2_ref[r, pl.ds(g, 16)]
                dv = d2_ref[r, pl.ds(g, 16)]
                ev = iota + r * 128 + g
                acc = cbv
                for k in range(DE):
                    kc = jnp.full((16,), k, i32)
                    acc = acc + plsc.load_gather(ea_v, [ev, kc]) * uvs[k]
                a_s = plsc.load_gather(sd_v, [sv + sv])
                a_d = plsc.load_gather(sd_v, [dv + dv + ones16i])
                lg = a_s + a_d + acc
                lg = jnp.maximum(lg, lg * 0.2)
                pe = jnp.exp(lg)
                plsc.addupdate_scatter(dloc, [dv], pe)
                for k in range(H):
                    kc = jnp.full((16,), k, i32)
                    v = plsc.load_gather(rb_ref, [ev, kc])
                    plsc.store_scatter(rb_ref, [ev, kc], v * pe)

    def scatter_fire(slot):
        for j in range(2):
            pltpu.async_copy(rbb[slot].at[pl.ds(j * 128, 128), :],
                             osh.at[d2b[slot].at[j]], tsems[slot], add=True)

    def scatter_wait(slot):
        for j in range(2):
            pltpu.make_async_copy(rbb[slot].at[pl.ds(j * 128, 128), :],
                                  osh.at[d2b[slot].at[j]],
                                  tsems[slot]).wait()

    fire(0, 0)
    fire(1, 1)

    @pl.loop(0, SC_ - 1, step=2)
    def _(it0):
        wait(0, it0)
        compute(0)
        scatter_fire(0)
        wait(1, it0 + 1)
        compute(1)
        scatter_fire(1)
        scatter_wait(0)
        fire(0, it0 + 2)
        scatter_wait(1)

        @pl.when(it0 + 3 < SC_)
        def _():
            fire(1, it0 + 3)

    wait(0, SC_ - 1)
    compute(0)
    scatter_fire(0)
    scatter_wait(0)

    @pl.when(w < XC)
    def _():
        xoff = (NWK * SC_ + w) * CC
        pltpu.sync_copy(eir_hbm.at[0, pl.ds((NWK * SC_ + w) * 2, 2), :], s21)
        pltpu.sync_copy(eir_hbm.at[1, pl.ds((NWK * SC_ + w) * 2, 2), :], d21)
        pltpu.sync_copy(ea_hbm.at[pl.ds(xoff, CC), :], ea1)
        for j in range(2):
            pltpu.sync_copy(h_hbm.at[s21.at[j]],
                            rb1.at[pl.ds(j * 128, 128), :])
        compute(1)
        scatter_fire(1)
        scatter_wait(1)

    pltpu.sync_copy(dloc, dn_hbm.at[c, s])

    plsc.subcore_barrier()
    for q, nrow in ((0, 256), (256, 256), (512, 128)):
        pltpu.sync_copy(osh.at[pl.ds(s * RPT + q, nrow), :],
                        rb0.at[pl.ds(0, nrow), :])
        pltpu.sync_copy(rb0.at[pl.ds(0, nrow), :],
                        out_hbm.at[c, pl.ds(s * RPT + q, nrow), :])


@functools.partial(
    pl.kernel,
    out_type=jax.ShapeDtypeStruct((E,), f32),
    mesh=_mesh,
    compiler_params=_sc_params,
    scratch_types=[
        pltpu.VMEM((SC_ * 2, 128), i32),
        pltpu.VMEM((SC_ * 2, 128), i32),
        pltpu.VMEM((CC, H), f32),
        pltpu.VMEM((CC, H), f32),
        pltpu.VMEM((H, 16), f32),
        pltpu.VMEM((1, 16), f32),
        pltpu.VMEM((SC_ * CC,), f32),
        pltpu.VMEM((2, 128), i32),
        pltpu.VMEM((2, 128), i32),
        pltpu.VMEM((CC,), f32),
        pltpu.SemaphoreType.DMA,
        pltpu.SemaphoreType.DMA,
        pltpu.SemaphoreType.DMA,
        pltpu.SemaphoreType.DMA,
    ],
)
def _pass_c(eir_hbm, gs_hbm, gd_hbm, w2bc_hbm, b2bc_hbm, sc_hbm,
            src2, dst2, fb0, fb1, w2_v, b2_v, sc_v,
            xsrc2, xdst2, xsc, sg0, sg1, sd0, sd1):
    c = lax.axis_index("c")
    s = lax.axis_index("s")
    w = s * 2 + c
    span0 = w * SC_
    iota = lax.iota(i32, 16)
    fbb = [fb0, fb1]
    sgb = [sg0, sg1]
    sdb = [sd0, sd1]
    pltpu.sync_copy(w2bc_hbm, w2_v)
    pltpu.sync_copy(b2bc_hbm, b2_v)
    pltpu.sync_copy(eir_hbm.at[0, pl.ds(span0 * 2, SC_ * 2), :], src2)
    pltpu.sync_copy(eir_hbm.at[1, pl.ds(span0 * 2, SC_ * 2), :], dst2)
    bias = b2_v[0]

    def fire_gs(slot, it):
        for j in range(2):
            pltpu.async_copy(gs_hbm.at[src2.at[it * 2 + j]],
                             fbb[slot].at[pl.ds(j * 128, 128), :], sgb[slot])

    def wait_gs(slot, it):
        for j in range(2):
            pltpu.make_async_copy(gs_hbm.at[src2.at[it * 2 + j]],
                                  fbb[slot].at[pl.ds(j * 128, 128), :],
                                  sgb[slot]).wait()

    def fire_gd(slot, it):
        for j in range(2):
            pltpu.async_copy(gd_hbm.at[dst2.at[it * 2 + j]],
                             fbb[slot].at[pl.ds(j * 128, 128), :], sdb[slot],
                             add=True)

    def wait_gd(slot, it):
        for j in range(2):
            pltpu.make_async_copy(gd_hbm.at[dst2.at[it * 2 + j]],
                                  fbb[slot].at[pl.ds(j * 128, 128), :],
                                  sdb[slot]).wait()

    def compute(slot, out_ref, base):
        fbuf = fbb[slot]

        @pl.loop(0, CC, step=16)
        def _(g):
            ev = iota + g
            acc = bias
            for k in range(H):
                kc = jnp.full((16,), k, i32)
                t = plsc.load_gather(fbuf, [ev, kc])
                acc = acc + jnp.maximum(t, 0.0) * w2_v[k]
            out_ref[pl.ds(base + g, 16)] = 1.0 / (1.0 + jnp.exp(-acc))

    fire_gs(0, 0)
    fire_gs(1, 1)
    wait_gs(0, 0)
    fire_gd(0, 0)

    @pl.loop(0, SC_ - 1, step=2)
    def _(it0):
        wait_gs(1, it0 + 1)
        fire_gd(1, it0 + 1)
        wait_gd(0, it0)
        compute(0, sc_v, it0 * CC)
        fire_gs(0, it0 + 2)
        wait_gd(1, it0 + 1)
        compute(1, sc_v, (it0 + 1) * CC)

        @pl.when(it0 + 3 < SC_)
        def _():
            fire_gs(1, it0 + 3)

        wait_gs(0, it0 + 2)
        fire_gd(0, it0 + 2)

    wait_gd(0, SC_ - 1)
    compute(0, sc_v, (SC_ - 1) * CC)
    pltpu.sync_copy(sc_v, sc_hbm.at[pl.ds(span0 * CC, SC_ * CC)])

    @pl.when(w < XC)
    def _():
        xoff = (NWK * SC_ + w) * CC
        pltpu.sync_copy(eir_hbm.at[0, pl.ds((NWK * SC_ + w) * 2, 2), :], xsrc2)
        pltpu.sync_copy(eir_hbm.at[1, pl.ds((NWK * SC_ + w) * 2, 2), :], xdst2)
        for j in range(2):
            pltpu.async_copy(gs_hbm.at[xsrc2.at[j]],
                             fb0.at[pl.ds(j * 128, 128), :], sg0)
            pltpu.make_async_copy(gs_hbm.at[xsrc2.at[j]],
                                  fb0.at[pl.ds(j * 128, 128), :], sg0).wait()
            pltpu.async_copy(gd_hbm.at[xdst2.at[j]],
                             fb0.at[pl.ds(j * 128, 128), :], sd0, add=True)
            pltpu.make_async_copy(gd_hbm.at[xdst2.at[j]],
                                  fb0.at[pl.ds(j * 128, 128), :], sd0).wait()
        compute(0, xsc, 0)
        pltpu.sync_copy(xsc, sc_hbm.at[pl.ds(xoff, CC)])


def kernel(x, edge_index, edge_attr, We, be, W1, We1, a_s1, a_d1, a_e1, b1,
           W2, We2, a_s2, a_d2, a_e2, b2, Ws1, bs1, Ws2, bs2):
    x_p = jnp.pad(x, ((0, NP - N), (0, 0)))
    A1 = jnp.stack([a_s1, a_d1], axis=1)
    A2 = jnp.stack([a_s2, a_d2], axis=1)
    eir = edge_index.reshape(2, ER, 128)

    h1, sd1, ubc1, ubc2, cbc = _tc0(
        x_p, W1, A1, We, be[None, :], We1, a_e1[:, None], We2, a_e2[:, None])

    o1, dn1 = _pass_ab(eir, edge_attr, sd1.reshape(-1), ubc1, cbc[0:1], h1)

    h2, sd2 = _tc1(o1, dn1.reshape(2, 16, NP // 128, 128), b1[None, :],
                   W2, A2)

    o2, dn2 = _pass_ab(eir, edge_attr, sd2.reshape(-1), ubc2, cbc[1:2], h2)

    gs, gd, w2bc, b2bc = _tc2(
        o2, dn2.reshape(2, 16, NP // 128, 128), b2[None, :], Ws1, bs1[None, :],
        Ws2, bs2[:, None])

    return _pass_c(eir, gs, gd, w2bc, b2bc)
